# baseline (speedup 1.0000x reference)
import functools

import jax
import jax.numpy as jnp
from jax import lax
from jax.experimental import pallas as pl
from jax.experimental.pallas import tpu as pltpu
from jax.experimental.pallas import tpu_sc as plsc

BATCH = 16384
EMBED_K = 32
NUM_WORKERS = 32
B_PER_W = BATCH // NUM_WORKERS
LANES = 16
PACK = 128 // EMBED_K
N_LINES = 1000000 // PACK


def _sc_fused(p_u, p_i, user_idx, item_idx):
    mesh = plsc.VectorSubcoreMesh(core_axis_name="c", subcore_axis_name="s")
    emb_t = jax.ShapeDtypeStruct((EMBED_K, BATCH), jnp.float32)
    dot_t = jax.ShapeDtypeStruct((BATCH,), jnp.float32)

    @functools.partial(
        pl.kernel,
        mesh=mesh,
        out_type=[dot_t, emb_t, emb_t],
        compiler_params=pltpu.CompilerParams(needs_layout_passes=False),
        scratch_types=[
            pltpu.VMEM((B_PER_W,), jnp.int32),
            pltpu.VMEM((B_PER_W,), jnp.int32),
            pltpu.VMEM((B_PER_W,), jnp.int32),
            pltpu.VMEM((B_PER_W, 128), jnp.float32),
            pltpu.VMEM((EMBED_K, B_PER_W), jnp.float32),
            pltpu.VMEM((EMBED_K, B_PER_W), jnp.float32),
            pltpu.VMEM((B_PER_W,), jnp.float32),
            pltpu.SemaphoreType.DMA,
        ],
    )
    def fused_kernel(u_tab, i_tab, u_idx, i_idx, dot_out, uc_out, ic_out,
                     idx_v, g_v, idx2_v, lines_v, s_u, s_i, dot_v, sem):
        wid = lax.axis_index("s") * 2 + lax.axis_index("c")
        base = wid * B_PER_W

        iota = lax.iota(jnp.int32, LANES)

        def gather_extract(tab, idx_hbm, s_out):
            pltpu.sync_copy(idx_hbm.at[pl.ds(base, B_PER_W)], idx_v)

            @pl.loop(0, B_PER_W, step=LANES)
            def _(j):
                sl = pl.ds(j, LANES)
                g_v[sl] = lax.shift_right_logical(idx_v[sl], 2)
            pltpu.async_copy(tab.at[g_v], lines_v, sem).wait()

            @pl.loop(0, B_PER_W, step=LANES)
            def _(j):
                sl = pl.ds(j, LANES)
                r_vec = idx_v[sl]
                o_vec = r_vec & (PACK - 1)
                row_vec = iota + j
                for c in range(EMBED_K):
                    s_out[c, sl] = plsc.load_gather(
                        lines_v, [row_vec, o_vec + PACK * c])

        gather_extract(u_tab, u_idx, s_u)
        gather_extract(i_tab, i_idx, s_i)

        @pl.loop(0, B_PER_W, step=LANES)
        def _(j):
            sl = pl.ds(j, LANES)
            u_sum = s_u[0, sl]
            i_sum = s_i[0, sl]
            for c in range(1, EMBED_K):
                u_sum = u_sum + s_u[c, sl]
                i_sum = i_sum + s_i[c, sl]
            u_mean = u_sum * (1.0 / EMBED_K)
            i_mean = i_sum * (1.0 / EMBED_K)
            acc = jnp.zeros((LANES,), jnp.float32)
            for c in range(EMBED_K):
                u_cent = s_u[c, sl] - u_mean
                i_cent = s_i[c, sl] - i_mean
                s_u[c, sl] = u_cent
                s_i[c, sl] = i_cent
                acc = acc + u_cent * i_cent
            dot_v[sl] = acc

        pltpu.sync_copy(s_u, uc_out.at[:, pl.ds(base, B_PER_W)])
        pltpu.sync_copy(s_i, ic_out.at[:, pl.ds(base, B_PER_W)])
        pltpu.sync_copy(dot_v, dot_out.at[pl.ds(base, B_PER_W)])

    return fused_kernel(p_u, p_i, user_idx, item_idx)


PACK_BLK = 4096
PACK_GRID = -(-1000000 // PACK_BLK)


def _tc_pack_body(t_ref, p_ref):
    x = t_ref[...]
    x3 = x.reshape(EMBED_K, PACK_BLK // PACK, PACK)
    p_ref[...] = x3.swapaxes(0, 1).reshape(PACK_BLK // PACK, 128)


def _tc_pack(tab_t):
    return pl.pallas_call(
        _tc_pack_body,
        grid=(PACK_GRID,),
        in_specs=[pl.BlockSpec((EMBED_K, PACK_BLK), lambda k: (0, k))],
        out_specs=pl.BlockSpec((PACK_BLK // PACK, 128), lambda k: (k, 0)),
        out_shape=jax.ShapeDtypeStruct((N_LINES, 128), jnp.float32),
    )(tab_t)


def kernel(x, user_table, item_table, scale_param):
    user_idx = x[:, 0]
    item_idx = x[:, 1]
    p_u = _tc_pack(user_table.T)
    p_i = _tc_pack(item_table.T)
    dot, uc_t, ic_t = _sc_fused(p_u, p_i, user_idx, item_idx)
    return (dot[:, None], uc_t.T, ic_t.T)


# speedup vs baseline: 8.4353x; 8.4353x over previous
import functools

import jax
import jax.numpy as jnp
from jax import lax
from jax.experimental import pallas as pl
from jax.experimental.pallas import tpu as pltpu
from jax.experimental.pallas import tpu_sc as plsc

BATCH = 16384
EMBED_K = 32
NUM_WORKERS = 32
B_PER_W = BATCH // NUM_WORKERS
LANES = 16
PACK = 128 // EMBED_K
PACK_BLK = 4096
PACK_SUB = PACK_BLK // PACK
PACK_GRID = -(-1000000 // PACK_BLK)
N_LINES = PACK_GRID * PACK_SUB


def _sc_fused(p_u, p_i, user_idx, item_idx):
    mesh = plsc.VectorSubcoreMesh(core_axis_name="c", subcore_axis_name="s")
    emb_t = jax.ShapeDtypeStruct((EMBED_K, BATCH), jnp.float32)
    dot_t = jax.ShapeDtypeStruct((BATCH,), jnp.float32)

    @functools.partial(
        pl.kernel,
        mesh=mesh,
        out_type=[dot_t, emb_t, emb_t],
        compiler_params=pltpu.CompilerParams(needs_layout_passes=False),
        scratch_types=[
            pltpu.VMEM((B_PER_W,), jnp.int32),
            pltpu.VMEM((B_PER_W,), jnp.int32),
            pltpu.VMEM((B_PER_W,), jnp.int32),
            pltpu.VMEM((B_PER_W, 128), jnp.float32),
            pltpu.VMEM((EMBED_K, B_PER_W), jnp.float32),
            pltpu.VMEM((EMBED_K, B_PER_W), jnp.float32),
            pltpu.VMEM((B_PER_W,), jnp.float32),
            pltpu.SemaphoreType.DMA,
        ],
    )
    def fused_kernel(u_tab, i_tab, u_idx, i_idx, dot_out, uc_out, ic_out,
                     idx_v, g_v, idx2_v, lines_v, s_u, s_i, dot_v, sem):
        wid = lax.axis_index("s") * 2 + lax.axis_index("c")
        base = wid * B_PER_W

        iota = lax.iota(jnp.int32, LANES)

        def gather_extract(tab, idx_hbm, s_out):
            pltpu.sync_copy(idx_hbm.at[pl.ds(base, B_PER_W)], idx_v)

            @pl.loop(0, B_PER_W, step=LANES)
            def _(j):
                sl = pl.ds(j, LANES)
                r_vec = idx_v[sl]
                g_v[sl] = lax.shift_left(
                    lax.shift_right_logical(r_vec, 12), 10) + (r_vec & 1023)
            pltpu.async_copy(tab.at[g_v], lines_v, sem).wait()

            @pl.loop(0, B_PER_W, step=LANES)
            def _(j):
                sl = pl.ds(j, LANES)
                r_vec = idx_v[sl]
                o_vec = (lax.shift_right_logical(r_vec, 10) & 3) * EMBED_K
                row_vec = iota + j
                for c in range(EMBED_K):
                    s_out[c, sl] = plsc.load_gather(
                        lines_v, [row_vec, o_vec + c])

        gather_extract(u_tab, u_idx, s_u)
        gather_extract(i_tab, i_idx, s_i)

        @pl.loop(0, B_PER_W, step=LANES)
        def _(j):
            sl = pl.ds(j, LANES)
            u_sum = s_u[0, sl]
            i_sum = s_i[0, sl]
            for c in range(1, EMBED_K):
                u_sum = u_sum + s_u[c, sl]
                i_sum = i_sum + s_i[c, sl]
            u_mean = u_sum * (1.0 / EMBED_K)
            i_mean = i_sum * (1.0 / EMBED_K)
            acc = jnp.zeros((LANES,), jnp.float32)
            for c in range(EMBED_K):
                u_cent = s_u[c, sl] - u_mean
                i_cent = s_i[c, sl] - i_mean
                s_u[c, sl] = u_cent
                s_i[c, sl] = i_cent
                acc = acc + u_cent * i_cent
            dot_v[sl] = acc

        pltpu.sync_copy(s_u, uc_out.at[:, pl.ds(base, B_PER_W)])
        pltpu.sync_copy(s_i, ic_out.at[:, pl.ds(base, B_PER_W)])
        pltpu.sync_copy(dot_v, dot_out.at[pl.ds(base, B_PER_W)])

    return fused_kernel(p_u, p_i, user_idx, item_idx)


def _tc_pack_body(t_ref, p_ref):
    x = t_ref[...]
    p_ref[...] = jnp.concatenate(
        [x[:, a * PACK_SUB:(a + 1) * PACK_SUB].T for a in range(PACK)], axis=1)


def _tc_pack(tab_t):
    return pl.pallas_call(
        _tc_pack_body,
        grid=(PACK_GRID,),
        in_specs=[pl.BlockSpec((EMBED_K, PACK_BLK), lambda k: (0, k))],
        out_specs=pl.BlockSpec((PACK_SUB, 128), lambda k: (k, 0)),
        out_shape=jax.ShapeDtypeStruct((N_LINES, 128), jnp.float32),
    )(tab_t)


def kernel(x, user_table, item_table, scale_param):
    user_idx = x[:, 0]
    item_idx = x[:, 1]
    p_u = _tc_pack(user_table.T)
    p_i = _tc_pack(item_table.T)
    dot, uc_t, ic_t = _sc_fused(p_u, p_i, user_idx, item_idx)
    return (dot[:, None], uc_t.T, ic_t.T)
